# baseline (speedup 1.0000x reference)
import jax
import jax.numpy as jnp
from jax.experimental import pallas as pl
from jax.experimental.pallas import tpu as pltpu


def _xw_kernel(x_ref, w_ref, o_ref):
    o_ref[...] = jnp.dot(x_ref[...], w_ref[...],
                         preferred_element_type=jnp.float32)


def _layer1_kernel(adj_ref, u_ref, b1_ref, w2_ref, w_out_ref):
    v = jnp.dot(adj_ref[...], u_ref[...], preferred_element_type=jnp.float32)
    h = jnp.maximum(v + b1_ref[...], 0.0)
    w_out_ref[...] = jnp.dot(h, w2_ref[...],
                             preferred_element_type=jnp.float32)


def _layer2_kernel(adj_ref, w_ref, b2_ref, o_ref):
    o_ref[...] = jnp.dot(adj_ref[...], w_ref[...],
                         preferred_element_type=jnp.float32) + b2_ref[...]


def kernel(x, adj, W1, b1, W2, b2):
    N, IN = x.shape
    HID = W1.shape[1]
    OUT = W2.shape[1]
    BMX = 1000
    BM = 400
    OUTP = 128

    W2p = jnp.zeros((HID, OUTP), W2.dtype).at[:, :OUT].set(W2)
    b2p = jnp.zeros((1, OUTP), b2.dtype).at[0, :OUT].set(b2)
    b1r = b1.reshape(1, HID)

    u = pl.pallas_call(
        _xw_kernel,
        grid=(N // BMX,),
        in_specs=[pl.BlockSpec((BMX, IN), lambda i: (i, 0)),
                  pl.BlockSpec((IN, HID), lambda i: (0, 0))],
        out_specs=pl.BlockSpec((BMX, HID), lambda i: (i, 0)),
        out_shape=jax.ShapeDtypeStruct((N, HID), jnp.float32),
        compiler_params=pltpu.CompilerParams(
            dimension_semantics=("parallel",)),
    )(x, W1)

    w = pl.pallas_call(
        _layer1_kernel,
        grid=(N // BM,),
        in_specs=[pl.BlockSpec((BM, N), lambda i: (i, 0)),
                  pl.BlockSpec((N, HID), lambda i: (0, 0)),
                  pl.BlockSpec((1, HID), lambda i: (0, 0)),
                  pl.BlockSpec((HID, OUTP), lambda i: (0, 0))],
        out_specs=pl.BlockSpec((BM, OUTP), lambda i: (i, 0)),
        out_shape=jax.ShapeDtypeStruct((N, OUTP), jnp.float32),
        compiler_params=pltpu.CompilerParams(
            dimension_semantics=("arbitrary",)),
    )(adj, u, b1r, W2p)

    out = pl.pallas_call(
        _layer2_kernel,
        grid=(N // BM,),
        in_specs=[pl.BlockSpec((BM, N), lambda i: (i, 0)),
                  pl.BlockSpec((N, OUTP), lambda i: (0, 0)),
                  pl.BlockSpec((1, OUTP), lambda i: (0, 0))],
        out_specs=pl.BlockSpec((BM, OUTP), lambda i: (i, 0)),
        out_shape=jax.ShapeDtypeStruct((N, OUTP), jnp.float32),
        compiler_params=pltpu.CompilerParams(
            dimension_semantics=("arbitrary",)),
    )(adj, w, b2p)

    return out[:, :OUT]
